# baseline (speedup 1.0000x reference)
import functools

import jax
import jax.numpy as jnp
from jax import lax
from jax.experimental import pallas as pl
from jax.experimental.pallas import tpu as pltpu
from jax.experimental.pallas import tpu_sc as plsc

N = 8192
BI = 512
CH = 2048
NEG = float("-inf")

S = 6144
NSC = N - S
NWORK = 32
R = NSC // NWORK
G = R // 16
JT = 2048
NJT = N // JT


def _tc_body(row_ref, col_ref, adj_ref, out_ref):
    part = None
    for c in range(N // CH):
        sl = slice(c * CH, (c + 1) * CH)
        t = jnp.dot(row_ref[:, :], col_ref[:, sl],
                    preferred_element_type=jnp.float32)
        adj = adj_ref[:, sl]
        for s in range(CH // 128):
            ssl = slice(s * 128, (s + 1) * 128)
            piece = jnp.where(adj[:, ssl] != 0, t[:, ssl], NEG)
            part = piece if part is None else jnp.maximum(part, piece)

    acc = jnp.max(part, axis=1, keepdims=True)
    d2 = acc + row_ref[:, 6:7]
    maxd = jnp.sqrt(jnp.maximum(d2, 0.0))
    prev = row_ref[:, 4:5]
    phimean = row_ref[:, 5:6]
    out_ref[:, :] = 0.5 * (prev + maxd * phimean)


def _sc_body(cols_hbm, p0_hbm, p1_hbm, p2_hbm, p3_hbm, p4_hbm, p5_hbm,
             adj_hbm, out_hbm,
             cols_v, p0_v, p1_v, p2_v, p3_v, p4_v, p5_v, abuf, outv):
    core = lax.axis_index("c")
    sub = lax.axis_index("s")
    wid = sub * 2 + core
    base = wid * R

    pltpu.sync_copy(cols_hbm, cols_v)
    pltpu.sync_copy(p0_hbm.at[pl.ds(base, R)], p0_v)
    pltpu.sync_copy(p1_hbm.at[pl.ds(base, R)], p1_v)
    pltpu.sync_copy(p2_hbm.at[pl.ds(base, R)], p2_v)
    pltpu.sync_copy(p3_hbm.at[pl.ds(base, R)], p3_v)
    pltpu.sync_copy(p4_hbm.at[pl.ds(base, R)], p4_v)
    pltpu.sync_copy(p5_hbm.at[pl.ds(base, R)], p5_v)

    def group(g, carry):
        gbase = g * 16
        c0v = p0_v[pl.ds(gbase, 16)]
        c1v = p1_v[pl.ds(gbase, 16)]
        c2v = p2_v[pl.ds(gbase, 16)]
        c0 = [c0v[rr] for rr in range(16)]
        c1 = [c1v[rr] for rr in range(16)]
        c2 = [c2v[rr] for rr in range(16)]

        accs = tuple(jnp.full((16,), NEG, jnp.float32) for _ in range(16))
        for jt in range(NJT):
            pltpu.sync_copy(
                adj_hbm.at[pl.ds(S + base + gbase, 16), pl.ds(jt * JT, JT)],
                abuf)

            def chunk(ci, accs):
                jg = jt * JT + ci * 16
                x0v = cols_v[0, pl.ds(jg, 16)]
                x1v = cols_v[1, pl.ds(jg, 16)]
                x2v = cols_v[2, pl.ds(jg, 16)]
                sqv = cols_v[3, pl.ds(jg, 16)]
                new = []
                for rr in range(16):
                    t = c0[rr] * x0v + c1[rr] * x1v + c2[rr] * x2v + sqv
                    av = abuf[rr, pl.ds(ci * 16, 16)]
                    tm = jnp.where(av != 0, t, NEG)
                    new.append(jnp.maximum(accs[rr], tm))
                return tuple(new)

            accs = lax.fori_loop(0, JT // 16, chunk, accs)

        laneidx = jnp.arange(16, dtype=jnp.int32)
        macc = jnp.full((16,), NEG, jnp.float32)
        for rr in range(16):
            elems = [accs[rr][l] for l in range(16)]
            while len(elems) > 1:
                elems = [jnp.maximum(elems[2 * k], elems[2 * k + 1])
                         for k in range(len(elems) // 2)]
            macc = jnp.where(laneidx == rr, elems[0], macc)

        sqiv = p3_v[pl.ds(gbase, 16)]
        prevv = p4_v[pl.ds(gbase, 16)]
        phiv = p5_v[pl.ds(gbase, 16)]
        d2 = jnp.maximum(macc + sqiv, 0.0)
        bits = lax.bitcast_convert_type(d2, jnp.int32)
        y = lax.bitcast_convert_type(jnp.int32(0x5F3759DF) - (bits >> 1),
                                     jnp.float32)
        for _ in range(3):
            y = y * (1.5 - 0.5 * d2 * y * y)
        maxd = jnp.where(d2 > 0.0, d2 * y, 0.0)
        outv[...] = 0.5 * (prevv + maxd * phiv)
        pltpu.sync_copy(outv, out_hbm.at[pl.ds(base + gbase, 16)])
        return carry

    lax.fori_loop(0, G, group, jnp.int32(0))


@jax.jit
def kernel(previous_inclusion_score, nodes, adjacency_matrix, W_phi, W_theta):
    w = W_theta[:, 0]
    wx = nodes * w[None, :]
    sq = jnp.sum(wx * wx, axis=1)
    phimean = jnp.mean(W_phi)

    zeros = jnp.zeros((N,), jnp.float32)
    ones = jnp.ones((N,), jnp.float32)
    phif = jnp.full((N,), phimean)
    rows = jnp.stack(
        [wx[:, 0], wx[:, 1], wx[:, 2], ones,
         previous_inclusion_score, phif, sq, zeros], axis=1)
    cols = jnp.stack(
        [-2.0 * wx[:, 0], -2.0 * wx[:, 1], -2.0 * wx[:, 2], sq,
         zeros, zeros, zeros, zeros], axis=0)

    tc_out = pl.pallas_call(
        _tc_body,
        grid=(S // BI,),
        in_specs=[
            pl.BlockSpec((BI, 8), lambda i: (i, 0)),
            pl.BlockSpec((8, N), lambda i: (0, 0)),
            pl.BlockSpec((BI, N), lambda i: (i, 0)),
        ],
        out_specs=pl.BlockSpec((BI, 1), lambda i: (i, 0)),
        out_shape=jax.ShapeDtypeStruct((S, 1), jnp.float32),
        compiler_params=pltpu.CompilerParams(
            dimension_semantics=("arbitrary",)),
    )(rows, cols, adjacency_matrix)

    cols_sc = jnp.stack([wx[:, 0], wx[:, 1], wx[:, 2], sq], axis=0)

    mesh = plsc.VectorSubcoreMesh(core_axis_name="c", subcore_axis_name="s")
    sc_out = pl.kernel(
        _sc_body,
        out_type=jax.ShapeDtypeStruct((NSC,), jnp.float32),
        mesh=mesh,
        scratch_types=[
            pltpu.VMEM((4, N), jnp.float32),
            pltpu.VMEM((R,), jnp.float32),
            pltpu.VMEM((R,), jnp.float32),
            pltpu.VMEM((R,), jnp.float32),
            pltpu.VMEM((R,), jnp.float32),
            pltpu.VMEM((R,), jnp.float32),
            pltpu.VMEM((R,), jnp.float32),
            pltpu.VMEM((16, JT), jnp.int32),
            pltpu.VMEM((16,), jnp.float32),
        ],
    )(cols_sc,
      -2.0 * wx[S:, 0], -2.0 * wx[S:, 1], -2.0 * wx[S:, 2],
      sq[S:], previous_inclusion_score[S:], phif[S:],
      adjacency_matrix)

    return jnp.concatenate([tc_out[:, 0], sc_out])


# speedup vs baseline: 1.3888x; 1.3888x over previous
import functools

import jax
import jax.numpy as jnp
from jax import lax
from jax.experimental import pallas as pl
from jax.experimental.pallas import tpu as pltpu
from jax.experimental.pallas import tpu_sc as plsc

N = 8192
BI = 512
CH = 2048
NEG = float("-inf")

S = 6144
NSC = N - S
NWORK = 32
R = NSC // NWORK
RG = 8
G = R // RG
JT = 2048
NJT = N // JT


def _tc_body(row_ref, col_ref, adj_ref, out_ref):
    part = None
    for c in range(N // CH):
        sl = slice(c * CH, (c + 1) * CH)
        t = jnp.dot(row_ref[:, :], col_ref[:, sl],
                    preferred_element_type=jnp.float32)
        adj = adj_ref[:, sl]
        for s in range(CH // 128):
            ssl = slice(s * 128, (s + 1) * 128)
            piece = jnp.where(adj[:, ssl] != 0, t[:, ssl], NEG)
            part = piece if part is None else jnp.maximum(part, piece)

    acc = jnp.max(part, axis=1, keepdims=True)
    d2 = acc + row_ref[:, 6:7]
    maxd = jnp.sqrt(jnp.maximum(d2, 0.0))
    prev = row_ref[:, 4:5]
    phimean = row_ref[:, 5:6]
    out_ref[:, :] = 0.5 * (prev + maxd * phimean)


def _sc_body(cols_hbm, p0_hbm, p1_hbm, p2_hbm, p3_hbm, p4_hbm, p5_hbm,
             adj_hbm, out_hbm,
             cols_v, p0_v, p1_v, p2_v, p3_v, p4_v, p5_v, abuf,
             outv, sem0, sem1):
    core = lax.axis_index("c")
    sub = lax.axis_index("s")
    wid = sub * 2 + core
    base = wid * R

    pltpu.sync_copy(cols_hbm, cols_v)
    pltpu.sync_copy(p0_hbm.at[pl.ds(base, R)], p0_v.at[pl.ds(0, R)])
    pltpu.sync_copy(p1_hbm.at[pl.ds(base, R)], p1_v.at[pl.ds(0, R)])
    pltpu.sync_copy(p2_hbm.at[pl.ds(base, R)], p2_v.at[pl.ds(0, R)])
    pltpu.sync_copy(p3_hbm.at[pl.ds(base, R)], p3_v.at[pl.ds(0, R)])
    pltpu.sync_copy(p4_hbm.at[pl.ds(base, R)], p4_v.at[pl.ds(0, R)])
    pltpu.sync_copy(p5_hbm.at[pl.ds(base, R)], p5_v.at[pl.ds(0, R)])

    sems = (sem0, sem1)

    def tile_copy(g, jt, buf):
        gc = jnp.minimum(g, G - 1)
        return pltpu.make_async_copy(
            adj_hbm.at[pl.ds(S + base + gc * RG, RG), pl.ds(jt * JT, JT)],
            abuf.at[buf], sems[buf])

    tile_copy(0, 0, 0).start()
    tile_copy(0, 1, 1).start()

    def group(g, carry):
        gbase = g * RG
        c0v = p0_v[pl.ds(gbase, 16)]
        c1v = p1_v[pl.ds(gbase, 16)]
        c2v = p2_v[pl.ds(gbase, 16)]
        c0 = [c0v[rr] for rr in range(RG)]
        c1 = [c1v[rr] for rr in range(RG)]
        c2 = [c2v[rr] for rr in range(RG)]

        accs = tuple(jnp.full((16,), NEG, jnp.float32) for _ in range(RG))
        for jt in range(NJT):
            buf = jt % 2
            tile_copy(g, jt, buf).wait()

            def chunk(ci, accs):
                jg = jt * JT + ci * 16
                x0v = cols_v[0, pl.ds(jg, 16)]
                x1v = cols_v[1, pl.ds(jg, 16)]
                x2v = cols_v[2, pl.ds(jg, 16)]
                sqv = cols_v[3, pl.ds(jg, 16)]
                new = []
                for rr in range(RG):
                    t = c0[rr] * x0v + c1[rr] * x1v + c2[rr] * x2v + sqv
                    av = abuf[buf, rr, pl.ds(ci * 16, 16)]
                    tm = jnp.where(av != 0, t, NEG)
                    new.append(jnp.maximum(accs[rr], tm))
                return tuple(new)

            accs = lax.fori_loop(0, JT // 16, chunk, accs)
            if jt < NJT - 2:
                tile_copy(g, jt + 2, buf).start()
            else:
                tile_copy(g + 1, jt + 2 - NJT, buf).start()

        laneidx = jnp.arange(16, dtype=jnp.int32)
        macc = jnp.full((16,), NEG, jnp.float32)
        for rr in range(RG):
            elems = [accs[rr][l] for l in range(16)]
            while len(elems) > 1:
                elems = [jnp.maximum(elems[2 * k], elems[2 * k + 1])
                         for k in range(len(elems) // 2)]
            macc = jnp.where(laneidx == rr, elems[0], macc)

        sqiv = p3_v[pl.ds(gbase, 16)]
        prevv = p4_v[pl.ds(gbase, 16)]
        phiv = p5_v[pl.ds(gbase, 16)]
        d2 = jnp.maximum(macc + sqiv, 0.0)
        bits = lax.bitcast_convert_type(d2, jnp.int32)
        y = lax.bitcast_convert_type(jnp.int32(0x5F3759DF) - (bits >> 1),
                                     jnp.float32)
        for _ in range(3):
            y = y * (1.5 - 0.5 * d2 * y * y)
        maxd = jnp.where(d2 > 0.0, d2 * y, 0.0)
        outv[...] = 0.5 * (prevv + maxd * phiv)
        pltpu.sync_copy(outv.at[pl.ds(0, RG)],
                        out_hbm.at[pl.ds(base + gbase, RG)])
        return carry

    lax.fori_loop(0, G, group, jnp.int32(0))
    tile_copy(G - 1, 0, 0).wait()
    tile_copy(G - 1, 1, 1).wait()


@jax.jit
def kernel(previous_inclusion_score, nodes, adjacency_matrix, W_phi, W_theta):
    w = W_theta[:, 0]
    wx = nodes * w[None, :]
    sq = jnp.sum(wx * wx, axis=1)
    phimean = jnp.mean(W_phi)

    zeros = jnp.zeros((N,), jnp.float32)
    ones = jnp.ones((N,), jnp.float32)
    phif = jnp.full((N,), phimean)
    rows = jnp.stack(
        [wx[:, 0], wx[:, 1], wx[:, 2], ones,
         previous_inclusion_score, phif, sq, zeros], axis=1)
    cols = jnp.stack(
        [-2.0 * wx[:, 0], -2.0 * wx[:, 1], -2.0 * wx[:, 2], sq,
         zeros, zeros, zeros, zeros], axis=0)

    tc_out = pl.pallas_call(
        _tc_body,
        grid=(S // BI,),
        in_specs=[
            pl.BlockSpec((BI, 8), lambda i: (i, 0)),
            pl.BlockSpec((8, N), lambda i: (0, 0)),
            pl.BlockSpec((BI, N), lambda i: (i, 0)),
        ],
        out_specs=pl.BlockSpec((BI, 1), lambda i: (i, 0)),
        out_shape=jax.ShapeDtypeStruct((S, 1), jnp.float32),
        compiler_params=pltpu.CompilerParams(
            dimension_semantics=("arbitrary",)),
    )(rows, cols, adjacency_matrix)

    cols_sc = jnp.stack([wx[:, 0], wx[:, 1], wx[:, 2], sq], axis=0)

    mesh = plsc.VectorSubcoreMesh(core_axis_name="c", subcore_axis_name="s")
    sc_out = pl.kernel(
        _sc_body,
        out_type=jax.ShapeDtypeStruct((NSC,), jnp.float32),
        mesh=mesh,
        scratch_types=[
            pltpu.VMEM((4, N), jnp.float32),
            pltpu.VMEM((R + 8,), jnp.float32),
            pltpu.VMEM((R + 8,), jnp.float32),
            pltpu.VMEM((R + 8,), jnp.float32),
            pltpu.VMEM((R + 8,), jnp.float32),
            pltpu.VMEM((R + 8,), jnp.float32),
            pltpu.VMEM((R + 8,), jnp.float32),
            pltpu.VMEM((2, RG, JT), jnp.int32),
            pltpu.VMEM((16,), jnp.float32),
            pltpu.SemaphoreType.DMA,
            pltpu.SemaphoreType.DMA,
        ],
    )(cols_sc,
      -2.0 * wx[S:, 0], -2.0 * wx[S:, 1], -2.0 * wx[S:, 2],
      sq[S:], previous_inclusion_score[S:], phif[S:],
      adjacency_matrix)

    return jnp.concatenate([tc_out[:, 0], sc_out])


# speedup vs baseline: 1.4368x; 1.0346x over previous
import numpy as np

import jax
import jax.numpy as jnp
from jax import lax
from jax.experimental import pallas as pl
from jax.experimental.pallas import tpu as pltpu
from jax.experimental.pallas import tpu_sc as plsc

N = 8192
BI = 512
CH = 2048
NEG = float("-inf")

S = 6144
NSC = N - S
NWORK = 32
R = NSC // NWORK
RG = 8
G = R // RG
JT = 2048
NJT = N // JT

_DN = (((0,), (0,)), ((), ()))


def _selector():
    ki = lax.broadcasted_iota(jnp.int32, (8, 3), 0)
    mi = lax.broadcasted_iota(jnp.int32, (8, 3), 1)
    return (ki == mi + 4).astype(jnp.float32)


def _tc_body(lhs_ref, rhs_ref, adj_ref, out_ref):
    lhs = lhs_ref[:, :]
    part = None
    for c in range(N // CH):
        sl = slice(c * CH, (c + 1) * CH)
        t = lax.dot_general(lhs, rhs_ref[:, sl], _DN,
                            preferred_element_type=jnp.float32)
        adj = adj_ref[:, sl]
        for s in range(CH // 128):
            ssl = slice(s * 128, (s + 1) * 128)
            piece = jnp.where(adj[:, ssl] != 0, t[:, ssl], NEG)
            part = piece if part is None else jnp.maximum(part, piece)

    aux = lax.dot_general(lhs, _selector(), _DN,
                          preferred_element_type=jnp.float32)
    acc = jnp.max(part, axis=1, keepdims=True)
    d2 = acc + aux[:, 2:3]
    maxd = jnp.sqrt(jnp.maximum(d2, 0.0))
    out_ref[:, :] = 0.5 * (aux[:, 0:1] + maxd * aux[:, 1:2])


def _sc_body(p_hbm, adj_hbm, out_hbm,
             cols_v, p0_v, p1_v, p2_v, p3_v, p4_v, p5_v, abuf,
             outv, sem0, sem1):
    core = lax.axis_index("c")
    sub = lax.axis_index("s")
    wid = sub * 2 + core
    base = wid * R

    pltpu.sync_copy(p_hbm.at[0], cols_v.at[0])
    pltpu.sync_copy(p_hbm.at[1], cols_v.at[1])
    pltpu.sync_copy(p_hbm.at[2], cols_v.at[2])
    pltpu.sync_copy(p_hbm.at[11], cols_v.at[3])
    hs = pl.ds(S + base, R)
    vs = pl.ds(0, R)
    pltpu.sync_copy(p_hbm.at[8, hs], p0_v.at[vs])
    pltpu.sync_copy(p_hbm.at[9, hs], p1_v.at[vs])
    pltpu.sync_copy(p_hbm.at[10, hs], p2_v.at[vs])
    pltpu.sync_copy(p_hbm.at[6, hs], p3_v.at[vs])
    pltpu.sync_copy(p_hbm.at[4, hs], p4_v.at[vs])
    pltpu.sync_copy(p_hbm.at[5, hs], p5_v.at[vs])

    sems = (sem0, sem1)

    def tile_copy(g, jt, buf):
        gc = jnp.minimum(g, G - 1)
        return pltpu.make_async_copy(
            adj_hbm.at[pl.ds(S + base + gc * RG, RG), pl.ds(jt * JT, JT)],
            abuf.at[buf], sems[buf])

    tile_copy(0, 0, 0).start()
    tile_copy(0, 1, 1).start()

    def group(g, carry):
        gbase = g * RG
        c0v = p0_v[pl.ds(gbase, 16)]
        c1v = p1_v[pl.ds(gbase, 16)]
        c2v = p2_v[pl.ds(gbase, 16)]
        c0 = [c0v[rr] for rr in range(RG)]
        c1 = [c1v[rr] for rr in range(RG)]
        c2 = [c2v[rr] for rr in range(RG)]

        accs = tuple(jnp.full((16,), NEG, jnp.float32) for _ in range(RG))
        for jt in range(NJT):
            buf = jt % 2
            tile_copy(g, jt, buf).wait()

            def chunk(ci, accs):
                jg = jt * JT + ci * 16
                x0v = cols_v[0, pl.ds(jg, 16)]
                x1v = cols_v[1, pl.ds(jg, 16)]
                x2v = cols_v[2, pl.ds(jg, 16)]
                sqv = cols_v[3, pl.ds(jg, 16)]
                new = []
                for rr in range(RG):
                    t = c0[rr] * x0v + c1[rr] * x1v + c2[rr] * x2v + sqv
                    av = abuf[buf, rr, pl.ds(ci * 16, 16)]
                    tm = jnp.where(av != 0, t, NEG)
                    new.append(jnp.maximum(accs[rr], tm))
                return tuple(new)

            accs = lax.fori_loop(0, JT // 16, chunk, accs)
            if jt < NJT - 2:
                tile_copy(g, jt + 2, buf).start()
            else:
                tile_copy(g + 1, jt + 2 - NJT, buf).start()

        laneidx = jnp.arange(16, dtype=jnp.int32)
        macc = jnp.full((16,), NEG, jnp.float32)
        for rr in range(RG):
            elems = [accs[rr][l] for l in range(16)]
            while len(elems) > 1:
                elems = [jnp.maximum(elems[2 * k], elems[2 * k + 1])
                         for k in range(len(elems) // 2)]
            macc = jnp.where(laneidx == rr, elems[0], macc)

        sqiv = p3_v[pl.ds(gbase, 16)]
        prevv = p4_v[pl.ds(gbase, 16)]
        phiv = p5_v[pl.ds(gbase, 16)]
        d2 = jnp.maximum(macc + sqiv, 0.0)
        bits = lax.bitcast_convert_type(d2, jnp.int32)
        y = lax.bitcast_convert_type(jnp.int32(0x5F3759DF) - (bits >> 1),
                                     jnp.float32)
        for _ in range(3):
            y = y * (1.5 - 0.5 * d2 * y * y)
        maxd = jnp.where(d2 > 0.0, d2 * y, 0.0)
        outv[...] = 0.5 * (prevv + maxd * phiv)
        pltpu.sync_copy(outv.at[pl.ds(0, RG)],
                        out_hbm.at[pl.ds(base + gbase, RG)])
        return carry

    lax.fori_loop(0, G, group, jnp.int32(0))
    tile_copy(G - 1, 0, 0).wait()
    tile_copy(G - 1, 1, 1).wait()


@jax.jit
def kernel(previous_inclusion_score, nodes, adjacency_matrix, W_phi, W_theta):
    wxT = nodes.T * W_theta[:, 0][:, None]
    sq = jnp.sum(wxT * wxT, axis=0)
    phimean = jnp.mean(W_phi)

    P = jnp.concatenate([
        wxT,
        jnp.ones((1, N), jnp.float32),
        previous_inclusion_score[None, :],
        jnp.broadcast_to(phimean, (1, N)),
        sq[None, :],
        jnp.zeros((1, N), jnp.float32),
        -2.0 * wxT,
        sq[None, :],
        jnp.zeros((4, N), jnp.float32),
    ], axis=0)

    tc_out = pl.pallas_call(
        _tc_body,
        grid=(S // BI,),
        in_specs=[
            pl.BlockSpec((8, BI), lambda i: (0, i)),
            pl.BlockSpec((8, N), lambda i: (1, 0)),
            pl.BlockSpec((BI, N), lambda i: (i, 0)),
        ],
        out_specs=pl.BlockSpec((BI, 1), lambda i: (i, 0)),
        out_shape=jax.ShapeDtypeStruct((S, 1), jnp.float32),
        compiler_params=pltpu.CompilerParams(
            dimension_semantics=("arbitrary",)),
    )(P, P, adjacency_matrix)

    mesh = plsc.VectorSubcoreMesh(core_axis_name="c", subcore_axis_name="s")
    sc_out = pl.kernel(
        _sc_body,
        out_type=jax.ShapeDtypeStruct((NSC,), jnp.float32),
        mesh=mesh,
        scratch_types=[
            pltpu.VMEM((4, N), jnp.float32),
            pltpu.VMEM((R + 8,), jnp.float32),
            pltpu.VMEM((R + 8,), jnp.float32),
            pltpu.VMEM((R + 8,), jnp.float32),
            pltpu.VMEM((R + 8,), jnp.float32),
            pltpu.VMEM((R + 8,), jnp.float32),
            pltpu.VMEM((R + 8,), jnp.float32),
            pltpu.VMEM((2, RG, JT), jnp.int32),
            pltpu.VMEM((16,), jnp.float32),
            pltpu.SemaphoreType.DMA,
            pltpu.SemaphoreType.DMA,
        ],
    )(P, adjacency_matrix)

    return jnp.concatenate([tc_out[:, 0], sc_out])


# speedup vs baseline: 1.4567x; 1.0138x over previous
import numpy as np

import jax
import jax.numpy as jnp
from jax import lax
from jax.experimental import pallas as pl
from jax.experimental.pallas import tpu as pltpu
from jax.experimental.pallas import tpu_sc as plsc

N = 8192
BI = 512
CH = 2048
NEG = float("-inf")

S = 6144
NSC = N - S
NWORK = 32
R = NSC // NWORK
RG = 8
G = R // RG
JT = 4096
NJT = N // JT

_DN = (((0,), (0,)), ((), ()))


def _selector():
    ki = lax.broadcasted_iota(jnp.int32, (8, 3), 0)
    mi = lax.broadcasted_iota(jnp.int32, (8, 3), 1)
    return (ki == mi + 4).astype(jnp.float32)


def _tc_body(lhs_ref, rhs_ref, adj_ref, out_ref):
    lhs = lhs_ref[:, :]
    part = None
    for c in range(N // CH):
        sl = slice(c * CH, (c + 1) * CH)
        t = lax.dot_general(lhs, rhs_ref[:, sl], _DN,
                            preferred_element_type=jnp.float32)
        adj = adj_ref[:, sl]
        for s in range(CH // 128):
            ssl = slice(s * 128, (s + 1) * 128)
            piece = jnp.where(adj[:, ssl] != 0, t[:, ssl], NEG)
            part = piece if part is None else jnp.maximum(part, piece)

    aux = lax.dot_general(lhs, _selector(), _DN,
                          preferred_element_type=jnp.float32)
    acc = jnp.max(part, axis=1, keepdims=True)
    d2 = acc + aux[:, 2:3]
    maxd = jnp.sqrt(jnp.maximum(d2, 0.0))
    res = 0.5 * (aux[:, 0:1] + maxd * aux[:, 1:2])
    out_ref[...] = res[:, 0]


def _sc_body(p_hbm, adj_hbm, out_hbm,
             cols_v, p0_v, p1_v, p2_v, p3_v, p4_v, p5_v, abuf,
             outv, sem0, sem1):
    core = lax.axis_index("c")
    sub = lax.axis_index("s")
    wid = sub * 2 + core
    base = wid * R

    pltpu.sync_copy(p_hbm.at[0], cols_v.at[0])
    pltpu.sync_copy(p_hbm.at[1], cols_v.at[1])
    pltpu.sync_copy(p_hbm.at[2], cols_v.at[2])
    pltpu.sync_copy(p_hbm.at[11], cols_v.at[3])
    hs = pl.ds(S + base, R)
    vs = pl.ds(0, R)
    pltpu.sync_copy(p_hbm.at[8, hs], p0_v.at[vs])
    pltpu.sync_copy(p_hbm.at[9, hs], p1_v.at[vs])
    pltpu.sync_copy(p_hbm.at[10, hs], p2_v.at[vs])
    pltpu.sync_copy(p_hbm.at[6, hs], p3_v.at[vs])
    pltpu.sync_copy(p_hbm.at[4, hs], p4_v.at[vs])
    pltpu.sync_copy(p_hbm.at[5, hs], p5_v.at[vs])

    sems = (sem0, sem1)

    def tile_copy(g, jt, buf):
        gc = jnp.minimum(g, G - 1)
        return pltpu.make_async_copy(
            adj_hbm.at[pl.ds(S + base + gc * RG, RG), pl.ds(jt * JT, JT)],
            abuf.at[buf], sems[buf])

    tile_copy(0, 0, 0).start()
    tile_copy(0, 1, 1).start()

    def group(g, carry):
        gbase = g * RG
        c0v = p0_v[pl.ds(gbase, 16)]
        c1v = p1_v[pl.ds(gbase, 16)]
        c2v = p2_v[pl.ds(gbase, 16)]
        c0 = [c0v[rr] for rr in range(RG)]
        c1 = [c1v[rr] for rr in range(RG)]
        c2 = [c2v[rr] for rr in range(RG)]

        accs = tuple(jnp.full((16,), NEG, jnp.float32) for _ in range(RG))
        for jt in range(NJT):
            buf = jt % 2
            tile_copy(g, jt, buf).wait()

            def chunk(ci, accs):
                jg = jt * JT + ci * 16
                x0v = cols_v[0, pl.ds(jg, 16)]
                x1v = cols_v[1, pl.ds(jg, 16)]
                x2v = cols_v[2, pl.ds(jg, 16)]
                sqv = cols_v[3, pl.ds(jg, 16)]
                new = []
                for rr in range(RG):
                    t = c0[rr] * x0v + c1[rr] * x1v + c2[rr] * x2v + sqv
                    av = abuf[buf, rr, pl.ds(ci * 16, 16)]
                    tm = jnp.where(av != 0, t, NEG)
                    new.append(jnp.maximum(accs[rr], tm))
                return tuple(new)

            accs = lax.fori_loop(0, JT // 16, chunk, accs)
            if jt < NJT - 2:
                tile_copy(g, jt + 2, buf).start()
            else:
                tile_copy(g + 1, jt + 2 - NJT, buf).start()

        laneidx = jnp.arange(16, dtype=jnp.int32)
        macc = jnp.full((16,), NEG, jnp.float32)
        for rr in range(RG):
            elems = [accs[rr][l] for l in range(16)]
            while len(elems) > 1:
                elems = [jnp.maximum(elems[2 * k], elems[2 * k + 1])
                         for k in range(len(elems) // 2)]
            macc = jnp.where(laneidx == rr, elems[0], macc)

        sqiv = p3_v[pl.ds(gbase, 16)]
        prevv = p4_v[pl.ds(gbase, 16)]
        phiv = p5_v[pl.ds(gbase, 16)]
        d2 = jnp.maximum(macc + sqiv, 0.0)
        bits = lax.bitcast_convert_type(d2, jnp.int32)
        y = lax.bitcast_convert_type(jnp.int32(0x5F3759DF) - (bits >> 1),
                                     jnp.float32)
        for _ in range(3):
            y = y * (1.5 - 0.5 * d2 * y * y)
        maxd = jnp.where(d2 > 0.0, d2 * y, 0.0)
        outv[...] = 0.5 * (prevv + maxd * phiv)
        pltpu.sync_copy(outv.at[pl.ds(0, RG)],
                        out_hbm.at[pl.ds(base + gbase, RG)])
        return carry

    lax.fori_loop(0, G, group, jnp.int32(0))
    tile_copy(G - 1, 0, 0).wait()
    tile_copy(G - 1, 1, 1).wait()


@jax.jit
def kernel(previous_inclusion_score, nodes, adjacency_matrix, W_phi, W_theta):
    wxT = nodes.T * W_theta[:, 0][:, None]
    sq = jnp.sum(wxT * wxT, axis=0)
    phimean = jnp.mean(W_phi)

    P = jnp.concatenate([
        wxT,
        jnp.ones((1, N), jnp.float32),
        previous_inclusion_score[None, :],
        jnp.broadcast_to(phimean, (1, N)),
        sq[None, :],
        jnp.zeros((1, N), jnp.float32),
        -2.0 * wxT,
        sq[None, :],
        jnp.zeros((4, N), jnp.float32),
    ], axis=0)

    tc_out = pl.pallas_call(
        _tc_body,
        grid=(S // BI,),
        in_specs=[
            pl.BlockSpec((8, BI), lambda i: (0, i)),
            pl.BlockSpec((8, N), lambda i: (1, 0)),
            pl.BlockSpec((BI, N), lambda i: (i, 0)),
        ],
        out_specs=pl.BlockSpec((BI,), lambda i: (i,)),
        out_shape=jax.ShapeDtypeStruct((S,), jnp.float32),
        compiler_params=pltpu.CompilerParams(
            dimension_semantics=("arbitrary",)),
    )(P, P, adjacency_matrix)

    mesh = plsc.VectorSubcoreMesh(core_axis_name="c", subcore_axis_name="s")
    sc_out = pl.kernel(
        _sc_body,
        out_type=jax.ShapeDtypeStruct((NSC,), jnp.float32),
        mesh=mesh,
        scratch_types=[
            pltpu.VMEM((4, N), jnp.float32),
            pltpu.VMEM((R + 8,), jnp.float32),
            pltpu.VMEM((R + 8,), jnp.float32),
            pltpu.VMEM((R + 8,), jnp.float32),
            pltpu.VMEM((R + 8,), jnp.float32),
            pltpu.VMEM((R + 8,), jnp.float32),
            pltpu.VMEM((R + 8,), jnp.float32),
            pltpu.VMEM((2, RG, JT), jnp.int32),
            pltpu.VMEM((16,), jnp.float32),
            pltpu.SemaphoreType.DMA,
            pltpu.SemaphoreType.DMA,
        ],
    )(P, adjacency_matrix)

    return jnp.concatenate([tc_out, sc_out])
